# baseline (speedup 1.0000x reference)
import jax
import jax.numpy as jnp
from jax import lax
from jax.experimental import pallas as pl
from jax.experimental.pallas import tpu as pltpu
from jax.experimental.pallas import tpu_sc as plsc

_B, _S, _D = 4, 2048, 512
_NC, _NS, _L = 2, 16, 16
_NW = _NC * _NS
_N = _B * _S
_SW = _S // _NW
_C = 16
_NG = _SW // _C
_NCHUNK = _B * _NG
_NB = 11


def _emb_body(x_hbm, table_hbm, pe_hbm, out_hbm,
              idx_v, pe_v, rows_v, isem, psem, gsem, ssem):
    wid = lax.axis_index("s") * _NC + lax.axis_index("c")
    s0 = wid * _SW

    pltpu.async_copy(pe_hbm.at[pl.ds(s0, _SW)], pe_v, psem)
    for b in range(_B):
        pltpu.async_copy(x_hbm.at[pl.ds(b * _S + s0, _SW)], idx_v.at[b], isem)
    for b in range(_B):
        pltpu.make_async_copy(x_hbm.at[pl.ds(b * _S + s0, _SW)],
                              idx_v.at[b], isem).wait()

    def chunk_coords(i):
        g, b = divmod(i, _B)
        return g, b

    def gather_copy(i):
        g, b = chunk_coords(i)
        return pltpu.make_async_copy(
            table_hbm.at[idx_v.at[b, pl.ds(g * _C, _C)]],
            rows_v.at[i % _NB], gsem.at[i % _NB])

    def store_copy(i):
        g, b = chunk_coords(i)
        return pltpu.make_async_copy(
            rows_v.at[i % _NB],
            out_hbm.at[pl.ds(b * _S + s0 + g * _C, _C)],
            ssem.at[i % _NB])

    for k in range(_B):
        gather_copy(k).start()
    pltpu.make_async_copy(pe_hbm.at[pl.ds(s0, _SW)], pe_v, psem).wait()

    for g in range(_NG):
        if g + 1 < _NG:
            for k in range(_B):
                j = (g + 1) * _B + k
                if j >= _NB:
                    store_copy(j - _NB).wait()
                gather_copy(j).start()

        for k in range(_B):
            gather_copy(g * _B + k).wait()

        @pl.loop(0, _C)
        def _row(r):
            for c in range(_D // _L):
                sl = pl.ds(c * _L, _L)
                p = pe_v[g * _C + r, sl]
                for k in range(_B):
                    plsc.addupdate(rows_v.at[(g * _B + k) % _NB, r, sl], p)

        for k in range(_B):
            store_copy(g * _B + k).start()

    for i in range(_NCHUNK - _NB, _NCHUNK):
        store_copy(i).wait()


def kernel(x, table, pe):
    mesh = plsc.VectorSubcoreMesh(core_axis_name="c", subcore_axis_name="s")
    out = pl.kernel(
        _emb_body,
        out_type=jax.ShapeDtypeStruct((_N, _D), jnp.float32),
        mesh=mesh,
        scratch_types=[
            pltpu.VMEM((_B, _SW), jnp.int32),
            pltpu.VMEM((_SW, _D), jnp.float32),
            pltpu.VMEM((_NB, _C, _D), jnp.float32),
            pltpu.SemaphoreType.DMA,
            pltpu.SemaphoreType.DMA,
            pltpu.SemaphoreType.DMA((_NB,)),
            pltpu.SemaphoreType.DMA((_NB,)),
        ],
    )(x.reshape(-1).astype(jnp.int32), table, pe)
    return out.reshape(_B, _S, _D)


# speedup vs baseline: 1.0049x; 1.0049x over previous
import jax
import jax.numpy as jnp
from jax import lax
from jax.experimental import pallas as pl
from jax.experimental.pallas import tpu as pltpu
from jax.experimental.pallas import tpu_sc as plsc

_B, _S, _D = 4, 2048, 512
_NC, _NS, _L = 2, 16, 16
_NW = _NC * _NS
_N = _B * _S
_SW = _S // _NW
_C = 16
_NG = _SW // _C
_NCHUNK = _B * _NG
_NB = 11


def _emb_body(x_hbm, table_hbm, pe_hbm, out_hbm,
              idx_v, pe_v, rows_v, isem, psem, gsem, ssem):
    wid = lax.axis_index("s") * _NC + lax.axis_index("c")
    s0 = wid * _SW

    pltpu.async_copy(pe_hbm.at[pl.ds(s0, _SW)], pe_v, psem)
    for b in range(_B):
        pltpu.async_copy(x_hbm.at[b, pl.ds(s0, _SW)], idx_v.at[b], isem)
    for b in range(_B):
        pltpu.make_async_copy(x_hbm.at[b, pl.ds(s0, _SW)],
                              idx_v.at[b], isem).wait()

    def chunk_coords(i):
        g, b = divmod(i, _B)
        return g, b

    def gather_copy(i):
        g, b = chunk_coords(i)
        return pltpu.make_async_copy(
            table_hbm.at[idx_v.at[b, pl.ds(g * _C, _C)]],
            rows_v.at[i % _NB], gsem.at[i % _NB])

    def store_copy(i):
        g, b = chunk_coords(i)
        return pltpu.make_async_copy(
            rows_v.at[i % _NB],
            out_hbm.at[b, pl.ds(s0 + g * _C, _C)],
            ssem.at[i % _NB])

    for k in range(_B):
        gather_copy(k).start()
    pltpu.make_async_copy(pe_hbm.at[pl.ds(s0, _SW)], pe_v, psem).wait()

    for g in range(_NG):
        if g + 1 < _NG:
            for k in range(_B):
                j = (g + 1) * _B + k
                if j >= _NB:
                    store_copy(j - _NB).wait()
                gather_copy(j).start()

        for k in range(_B):
            gather_copy(g * _B + k).wait()

        @pl.loop(0, _C)
        def _row(r):
            for c in range(_D // _L):
                sl = pl.ds(c * _L, _L)
                p = pe_v[g * _C + r, sl]
                for k in range(_B):
                    plsc.addupdate(rows_v.at[(g * _B + k) % _NB, r, sl], p)

        for k in range(_B):
            store_copy(g * _B + k).start()

    for i in range(_NCHUNK - _NB, _NCHUNK):
        store_copy(i).wait()


def kernel(x, table, pe):
    mesh = plsc.VectorSubcoreMesh(core_axis_name="c", subcore_axis_name="s")
    out = pl.kernel(
        _emb_body,
        out_type=jax.ShapeDtypeStruct((_B, _S, _D), jnp.float32),
        mesh=mesh,
        scratch_types=[
            pltpu.VMEM((_B, _SW), jnp.int32),
            pltpu.VMEM((_SW, _D), jnp.float32),
            pltpu.VMEM((_NB, _C, _D), jnp.float32),
            pltpu.SemaphoreType.DMA,
            pltpu.SemaphoreType.DMA,
            pltpu.SemaphoreType.DMA((_NB,)),
            pltpu.SemaphoreType.DMA((_NB,)),
        ],
    )(x, table, pe)
    return out
